# Initial kernel scaffold; baseline (speedup 1.0000x reference)
import jax
import jax.numpy as jnp
from jax.experimental import pallas as pl


def kernel(q, k, v):
    raise NotImplementedError("write your pallas kernel here")


# speedup vs baseline: 3.0436x; 3.0436x over previous
import functools

import jax
import jax.numpy as jnp
from jax.experimental import pallas as pl

_WS = 14
_TOPK = 8
_DH = 32
_EPS = 1e-6

_call = pl.pallas_call


def _elu1(x):
    return jnp.where(x > 0, x + 1.0, jnp.exp(jnp.where(x > 0, 0.0, x)))


def _stats_body(qw_ref, kw_ref, vw_ref, qm_ref, km_ref, vm_ref, s_ref, pd_ref,
                *, wb, hh, dh):
    for w in range(wb):
        qwin = qw_ref[0, w]
        kwin = kw_ref[0, w]
        vwin = vw_ref[0, w]
        kf = _elu1(kwin)
        p = jax.lax.dot_general(kf, vwin, (((0,), (0,)), ((), ())),
                                preferred_element_type=jnp.float32)
        for h in range(hh):
            pd_ref[0, w, h * dh:(h + 1) * dh, :] = (
                p[h * dh:(h + 1) * dh, h * dh:(h + 1) * dh])
        s_ref[0, w:w + 1, :] = jnp.sum(kf, axis=0, keepdims=True)
        qm_ref[0, w:w + 1, :] = jnp.mean(qwin, axis=0, keepdims=True)
        km_ref[0, w:w + 1, :] = jnp.mean(kwin, axis=0, keepdims=True)
        vm_ref[0, w:w + 1, :] = jnp.mean(vwin, axis=0, keepdims=True)


def _route_body(qm_ref, km_ref, vm_ref, mask_ref, ckv_ref, cs_ref,
                *, nw, topk, hh, dh):
    qm = qm_ref[0]
    km = km_ref[0]
    vm = vm_ref[0]
    sim = jax.lax.dot_general(qm, km, (((1,), (1,)), ((), ())),
                              preferred_element_type=jnp.float32)
    li = jax.lax.broadcasted_iota(jnp.int32, (nw, nw), 1)
    msum = jnp.zeros((nw, nw), jnp.float32)
    cur = sim
    for _ in range(topk):
        mx = jnp.max(cur, axis=1, keepdims=True)
        ismax = cur == mx
        first = jnp.min(jnp.where(ismax, li, nw), axis=1, keepdims=True)
        sel = li == first
        msum = msum + sel.astype(jnp.float32)
        cur = jnp.where(sel, -jnp.inf, cur)
    mask_ref[0] = msum
    kfm = _elu1(km)
    cp = jax.lax.dot_general(kfm, vm, (((0,), (0,)), ((), ())),
                             preferred_element_type=jnp.float32)
    for h in range(hh):
        ckv_ref[0, h * dh:(h + 1) * dh, :] = (
            cp[h * dh:(h + 1) * dh, h * dh:(h + 1) * dh])
    cs_ref[0, 0:1, :] = jnp.sum(kfm, axis=0, keepdims=True)


def _gather_body(mask_ref, pdf_ref, s_ref, ckvf_ref, cs_ref, kvg_ref, sg_ref):
    mask = mask_ref[0]
    kvg_ref[0] = jax.lax.dot_general(
        mask, pdf_ref[0], (((1,), (0,)), ((), ())),
        preferred_element_type=jnp.float32) + ckvf_ref[0]
    sg_ref[0] = jax.lax.dot_general(
        mask, s_ref[0], (((1,), (0,)), ((), ())),
        preferred_element_type=jnp.float32) + cs_ref[0]


def _attn_body(qw_ref, kv_ref, sg_ref, out_ref, *, wb, hh, dh):
    for w in range(wb):
        qf = _elu1(qw_ref[0, w])
        kv = kv_ref[0, w]
        sg = sg_ref[0, w:w + 1, :]
        for h in range(hh):
            qfh = qf[:, h * dh:(h + 1) * dh]
            num = jnp.dot(qfh, kv[h * dh:(h + 1) * dh, :],
                          preferred_element_type=jnp.float32)
            den = jnp.sum(qfh * sg[:, h * dh:(h + 1) * dh], axis=1,
                          keepdims=True) + _EPS
            out_ref[0, w, :, h * dh:(h + 1) * dh] = num / den


def kernel(q, k, v):
    b, d, H, W = q.shape
    ws = _WS
    m, n = H // ws, W // ws
    nw = m * n
    t = ws * ws
    hh = d // _DH
    dh = _DH
    topk = _TOPK
    f32 = jnp.float32

    def to_win(x):
        x = x.reshape(b, d, m, ws, n, ws)
        x = jnp.transpose(x, (0, 2, 4, 3, 5, 1))
        return x.reshape(b, nw, t, d)

    qw, kw, vw = to_win(q), to_win(k), to_win(v)

    wb = 8
    stats = _call(
        functools.partial(_stats_body, wb=wb, hh=hh, dh=dh),
        grid=(b, nw // wb),
        in_specs=[pl.BlockSpec((1, wb, t, d), lambda i, j: (i, j, 0, 0))] * 3,
        out_specs=[
            pl.BlockSpec((1, wb, d), lambda i, j: (i, j, 0)),
            pl.BlockSpec((1, wb, d), lambda i, j: (i, j, 0)),
            pl.BlockSpec((1, wb, d), lambda i, j: (i, j, 0)),
            pl.BlockSpec((1, wb, d), lambda i, j: (i, j, 0)),
            pl.BlockSpec((1, wb, d, dh), lambda i, j: (i, j, 0, 0)),
        ],
        out_shape=[
            jax.ShapeDtypeStruct((b, nw, d), f32),
            jax.ShapeDtypeStruct((b, nw, d), f32),
            jax.ShapeDtypeStruct((b, nw, d), f32),
            jax.ShapeDtypeStruct((b, nw, d), f32),
            jax.ShapeDtypeStruct((b, nw, d, dh), f32),
        ],
    )(qw, kw, vw)
    qm, km, vm, s_sum, pd = stats

    mask, ckv, cs = _call(
        functools.partial(_route_body, nw=nw, topk=topk, hh=hh, dh=dh),
        grid=(b,),
        in_specs=[pl.BlockSpec((1, nw, d), lambda i: (i, 0, 0))] * 3,
        out_specs=[
            pl.BlockSpec((1, nw, nw), lambda i: (i, 0, 0)),
            pl.BlockSpec((1, d, dh), lambda i: (i, 0, 0)),
            pl.BlockSpec((1, 1, d), lambda i: (i, 0, 0)),
        ],
        out_shape=[
            jax.ShapeDtypeStruct((b, nw, nw), f32),
            jax.ShapeDtypeStruct((b, d, dh), f32),
            jax.ShapeDtypeStruct((b, 1, d), f32),
        ],
    )(qm, km, vm)

    pdf = pd.reshape(b, nw, d * dh)
    ckvf = ckv.reshape(b, 1, d * dh)

    kvg, sg = _call(
        _gather_body,
        grid=(b,),
        in_specs=[
            pl.BlockSpec((1, nw, nw), lambda i: (i, 0, 0)),
            pl.BlockSpec((1, nw, d * dh), lambda i: (i, 0, 0)),
            pl.BlockSpec((1, nw, d), lambda i: (i, 0, 0)),
            pl.BlockSpec((1, 1, d * dh), lambda i: (i, 0, 0)),
            pl.BlockSpec((1, 1, d), lambda i: (i, 0, 0)),
        ],
        out_specs=[
            pl.BlockSpec((1, nw, d * dh), lambda i: (i, 0, 0)),
            pl.BlockSpec((1, nw, d), lambda i: (i, 0, 0)),
        ],
        out_shape=[
            jax.ShapeDtypeStruct((b, nw, d * dh), f32),
            jax.ShapeDtypeStruct((b, nw, d), f32),
        ],
    )(mask, pdf, s_sum, ckvf, cs)

    kv4 = kvg.reshape(b, nw, d, dh)

    msg = _call(
        functools.partial(_attn_body, wb=wb, hh=hh, dh=dh),
        grid=(b, nw // wb),
        in_specs=[
            pl.BlockSpec((1, wb, t, d), lambda i, j: (i, j, 0, 0)),
            pl.BlockSpec((1, wb, d, dh), lambda i, j: (i, j, 0, 0)),
            pl.BlockSpec((1, wb, d), lambda i, j: (i, j, 0)),
        ],
        out_specs=pl.BlockSpec((1, wb, t, d), lambda i, j: (i, j, 0, 0)),
        out_shape=jax.ShapeDtypeStruct((b, nw, t, d), f32),
    )(qw, kv4, sg)

    out = msg.reshape(b, m, n, ws, ws, d)
    out = jnp.transpose(out, (0, 5, 1, 3, 2, 4))
    return out.reshape(b, d, H, W)


# speedup vs baseline: 3.3678x; 1.1065x over previous
import functools

import jax
import jax.numpy as jnp
from jax.experimental import pallas as pl

_WS = 14
_TOPK = 8
_DH = 32
_EPS = 1e-6

_call = pl.pallas_call


def _elu1(x):
    return jnp.where(x > 0, x + 1.0, jnp.exp(jnp.where(x > 0, 0.0, x)))


def _blockdiag(p_aug, d, hh, dh):
    r_h = jax.lax.broadcasted_iota(jnp.int32, (d, d), 0) // dh
    c_h = jax.lax.broadcasted_iota(jnp.int32, (d, d), 1) // dh
    bd = p_aug[:, 0:d] * (r_h == c_h).astype(jnp.float32)
    rs_h = jax.lax.broadcasted_iota(jnp.int32, (d, hh), 0) // dh
    cs_h = jax.lax.broadcasted_iota(jnp.int32, (d, hh), 1)
    s = p_aug[:, d:d + 1] * (rs_h == cs_h).astype(jnp.float32)
    return jnp.concatenate([bd, s], axis=1)


def _stats_body(qw_ref, kw_ref, vw_ref, qm_ref, km_ref, vm_ref, pd_ref,
                *, wb, hh, dh, t, d):
    mean_row = jnp.full((1, t), 1.0 / t, dtype=jnp.float32)
    ones_row = jnp.ones((1, t), dtype=jnp.float32)
    for w in range(wb):
        qwin = qw_ref[0, w]
        kwin = kw_ref[0, w]
        vwin = vw_ref[0, w]
        kf = _elu1(kwin)
        v_aug = jnp.concatenate([vwin, ones_row], axis=0)
        p = jax.lax.dot_general(kf, v_aug, (((1,), (1,)), ((), ())),
                                preferred_element_type=jnp.float32)
        pd_ref[0, w] = _blockdiag(p, d, hh, dh)
        qk = jnp.concatenate([qwin, kwin], axis=0)
        hi = jax.lax.bitcast_convert_type(
            jax.lax.bitcast_convert_type(qk, jnp.int32)
            & jnp.int32(-65536), jnp.float32)
        rem = qk - hi
        mid = jax.lax.bitcast_convert_type(
            jax.lax.bitcast_convert_type(rem, jnp.int32)
            & jnp.int32(-65536), jnp.float32)
        lo = rem - mid
        cat = jnp.concatenate([hi, mid, lo, vwin], axis=0)
        r = jax.lax.dot_general(ones_row, cat, (((1,), (1,)), ((), ())),
                                preferred_element_type=jnp.float32)
        inv_t = jnp.float32(1.0 / t)
        qm_ref[0, w:w + 1, :] = (
            (r[:, 0:d] + r[:, 2 * d:3 * d]) + r[:, 4 * d:5 * d]) * inv_t
        km_ref[0, w:w + 1, :] = (
            (r[:, d:2 * d] + r[:, 3 * d:4 * d]) + r[:, 5 * d:6 * d]) * inv_t
        vm_ref[0, w:w + 1, :] = r[:, 6 * d:7 * d] * inv_t


def _route_body(qm_ref, km_ref, vm_ref, pdf_ref, kvg_ref, ckv_ref,
                *, nw, topk, hh, dh, d):
    qm = qm_ref[0]
    km = km_ref[0]
    vm = vm_ref[0]
    sim = jax.lax.dot_general(qm, km, (((1,), (1,)), ((), ())),
                              preferred_element_type=jnp.float32)
    li = jax.lax.broadcasted_iota(jnp.int32, (nw, nw), 1)
    msum = jnp.zeros((nw, nw), jnp.float32)
    cur = sim
    for _ in range(topk):
        mx = jnp.max(cur, axis=1, keepdims=True)
        ismax = cur == mx
        first = jnp.min(jnp.where(ismax, li, nw), axis=1, keepdims=True)
        sel = li == first
        msum = msum + sel.astype(jnp.float32)
        cur = jnp.where(sel, -jnp.inf, cur)
    kfm = _elu1(km)
    vm_aug = jnp.concatenate([vm, jnp.ones((nw, 1), jnp.float32)], axis=1)
    cp = jax.lax.dot_general(kfm, vm_aug, (((0,), (0,)), ((), ())),
                             preferred_element_type=jnp.float32)
    ckv_ref[0] = _blockdiag(cp, d, hh, dh)
    kvg_ref[0] = jax.lax.dot_general(
        msum, pdf_ref[0], (((1,), (0,)), ((), ())),
        preferred_element_type=jnp.float32)


def _attn_body(qw_ref, kv_ref, ckv_ref, out_ref, *, wb, hh, dh, d):
    ch = jax.lax.broadcasted_iota(jnp.int32, (hh, d), 1) // dh
    rh = jax.lax.broadcasted_iota(jnp.int32, (hh, d), 0)
    expand = (ch == rh).astype(jnp.float32)
    central = ckv_ref[0]
    for w in range(wb):
        qf = _elu1(qw_ref[0, w])
        r = jax.lax.dot_general(
            kv_ref[0, w] + central, qf, (((0,), (0,)), ((), ())),
            preferred_element_type=jnp.float32)
        den = jax.lax.dot_general(
            expand, r[d:d + hh, :], (((0,), (0,)), ((), ())),
            preferred_element_type=jnp.float32) + _EPS
        out_ref[0, w] = r[0:d, :] / den


def kernel(q, k, v):
    b, d, H, W = q.shape
    ws = _WS
    m, n = H // ws, W // ws
    nw = m * n
    t = ws * ws
    hh = d // _DH
    dh = _DH
    topk = _TOPK
    da = d + hh
    f32 = jnp.float32

    def to_win(x):
        x = x.reshape(b, d, m, ws, n, ws)
        x = jnp.transpose(x, (0, 2, 4, 1, 3, 5))
        return x.reshape(b, nw, d, t)

    qw, kw, vw = to_win(q), to_win(k), to_win(v)

    wbs = 16
    wba = 8
    stats = _call(
        functools.partial(_stats_body, wb=wbs, hh=hh, dh=dh, t=t, d=d),
        grid=(b, nw // wbs),
        in_specs=[pl.BlockSpec((1, wbs, d, t), lambda i, j: (i, j, 0, 0))] * 3,
        out_specs=[
            pl.BlockSpec((1, wbs, d), lambda i, j: (i, j, 0)),
            pl.BlockSpec((1, wbs, d), lambda i, j: (i, j, 0)),
            pl.BlockSpec((1, wbs, d), lambda i, j: (i, j, 0)),
            pl.BlockSpec((1, wbs, d, da), lambda i, j: (i, j, 0, 0)),
        ],
        out_shape=[
            jax.ShapeDtypeStruct((b, nw, d), f32),
            jax.ShapeDtypeStruct((b, nw, d), f32),
            jax.ShapeDtypeStruct((b, nw, d), f32),
            jax.ShapeDtypeStruct((b, nw, d, da), f32),
        ],
    )(qw, kw, vw)
    qm, km, vm, pd = stats

    pdf = pd.reshape(b, nw, d * da)

    kvg, ckv = _call(
        functools.partial(_route_body, nw=nw, topk=topk, hh=hh, dh=dh, d=d),
        grid=(b,),
        in_specs=[
            pl.BlockSpec((1, nw, d), lambda i: (i, 0, 0)),
            pl.BlockSpec((1, nw, d), lambda i: (i, 0, 0)),
            pl.BlockSpec((1, nw, d), lambda i: (i, 0, 0)),
            pl.BlockSpec((1, nw, d * da), lambda i: (i, 0, 0)),
        ],
        out_specs=[
            pl.BlockSpec((1, nw, d * da), lambda i: (i, 0, 0)),
            pl.BlockSpec((1, d, da), lambda i: (i, 0, 0)),
        ],
        out_shape=[
            jax.ShapeDtypeStruct((b, nw, d * da), f32),
            jax.ShapeDtypeStruct((b, d, da), f32),
        ],
    )(qm, km, vm, pdf)

    kv4 = kvg.reshape(b, nw, d, da)

    msg = _call(
        functools.partial(_attn_body, wb=wba, hh=hh, dh=dh, d=d),
        grid=(b, nw // wba),
        in_specs=[
            pl.BlockSpec((1, wba, d, t), lambda i, j: (i, j, 0, 0)),
            pl.BlockSpec((1, wba, d, da), lambda i, j: (i, j, 0, 0)),
            pl.BlockSpec((1, d, da), lambda i, j: (i, 0, 0)),
        ],
        out_specs=pl.BlockSpec((1, wba, d, t), lambda i, j: (i, j, 0, 0)),
        out_shape=jax.ShapeDtypeStruct((b, nw, d, t), f32),
    )(qw, kv4, ckv)

    out = msg.reshape(b, m, n, d, ws, ws)
    out = jnp.transpose(out, (0, 3, 1, 4, 2, 5))
    return out.reshape(b, d, H, W)
